# Initial kernel scaffold; baseline (speedup 1.0000x reference)
import jax
import jax.numpy as jnp
from jax.experimental import pallas as pl


def kernel(xs, scores, expert_weights):
    raise NotImplementedError("write your pallas kernel here")


# speedup vs baseline: 5.7759x; 5.7759x over previous
import functools

import jax
import jax.numpy as jnp
from jax.experimental import pallas as pl
from jax.experimental.pallas import tpu as pltpu

B, K, C, L = 16, 8, 128, 4096
TOPK = 4
LB = 4096


def _routing_body(s_ref, sel_ref, w_ref):
    s = s_ref[...]
    iota = jax.lax.broadcasted_iota(jnp.int32, (B, K), 1)
    cur = s
    vals = []
    idxs = []
    for _ in range(TOPK):
        m = jnp.max(cur, axis=1, keepdims=True)
        is_m = cur == m
        idx = jnp.min(jnp.where(is_m, iota, K), axis=1, keepdims=True)
        vals.append(m)
        idxs.append(idx)
        cur = jnp.where(iota == idx, -jnp.inf, cur)
    v = jnp.concatenate(vals, axis=1)
    i = jnp.concatenate(idxs, axis=1)
    w = v / (jnp.sum(v, axis=1, keepdims=True) + 1e-8)
    sel_ref[...] = i
    w_ref[...] = w


def _moe_body(sel_ref, w_ref, x_ref, ew_ref, o_ref):
    b = pl.program_id(0)
    t = pl.program_id(2)
    w = w_ref[b, t]
    y = jax.lax.dot_general(
        ew_ref[0], x_ref[0, 0],
        dimension_numbers=(((0,), (0,)), ((), ())),
        preferred_element_type=jnp.float32,
    )
    contrib = y * w

    @pl.when(t == 0)
    def _():
        o_ref[0] = contrib

    @pl.when(t != 0)
    def _():
        o_ref[0] = o_ref[0] + contrib


@jax.jit
def kernel(xs, scores, expert_weights):
    sel, w = pl.pallas_call(
        _routing_body,
        out_shape=(
            jax.ShapeDtypeStruct((B, TOPK), jnp.int32),
            jax.ShapeDtypeStruct((B, TOPK), jnp.float32),
        ),
    )(scores)

    n_l = L // LB
    grid = (B, n_l, TOPK)
    out = pl.pallas_call(
        _moe_body,
        grid_spec=pltpu.PrefetchScalarGridSpec(
            num_scalar_prefetch=2,
            grid=grid,
            in_specs=[
                pl.BlockSpec(
                    (1, 1, C, LB),
                    lambda b, l, t, sel_ref, w_ref: (b, sel_ref[b, t], 0, l),
                ),
                pl.BlockSpec(
                    (1, C, C),
                    lambda b, l, t, sel_ref, w_ref: (sel_ref[b, t], 0, 0),
                ),
            ],
            out_specs=pl.BlockSpec(
                (1, C, LB),
                lambda b, l, t, sel_ref, w_ref: (b, 0, l),
            ),
        ),
        out_shape=jax.ShapeDtypeStruct((B, C, L), jnp.float32),
    )(sel, w, xs, expert_weights)
    return out


# speedup vs baseline: 5.7876x; 1.0020x over previous
import functools

import jax
import jax.numpy as jnp
from jax.experimental import pallas as pl
from jax.experimental.pallas import tpu as pltpu

B, K, C, L = 16, 8, 128, 4096
TOPK = 4
LB = 4096


def _routing_body(s_ref, sel_ref, w_ref):
    s = s_ref[...]
    iota = jax.lax.broadcasted_iota(jnp.int32, (B, K), 1)
    cur = s
    vals = []
    idxs = []
    for _ in range(TOPK):
        m = jnp.max(cur, axis=1, keepdims=True)
        is_m = cur == m
        idx = jnp.min(jnp.where(is_m, iota, K), axis=1, keepdims=True)
        vals.append(m)
        idxs.append(idx)
        cur = jnp.where(iota == idx, -jnp.inf, cur)
    v = jnp.concatenate(vals, axis=1)
    i = jnp.concatenate(idxs, axis=1)
    w = v / (jnp.sum(v, axis=1, keepdims=True) + 1e-8)
    sel_ref[...] = i
    w_ref[...] = w


def _moe_body(sel_ref, w_ref, x_ref, ew_ref, o_ref):
    b = pl.program_id(0)
    t = pl.program_id(2)
    w = w_ref[b, t]
    ew = (ew_ref[0] * w).astype(jnp.bfloat16)
    x = x_ref[0, 0].astype(jnp.bfloat16)
    contrib = jax.lax.dot_general(
        ew, x,
        dimension_numbers=(((0,), (0,)), ((), ())),
        preferred_element_type=jnp.float32,
    )

    @pl.when(t == 0)
    def _():
        o_ref[0] = contrib

    @pl.when(t != 0)
    def _():
        o_ref[0] = o_ref[0] + contrib


@jax.jit
def kernel(xs, scores, expert_weights):
    sel, w = pl.pallas_call(
        _routing_body,
        out_shape=(
            jax.ShapeDtypeStruct((B, TOPK), jnp.int32),
            jax.ShapeDtypeStruct((B, TOPK), jnp.float32),
        ),
    )(scores)

    n_l = L // LB
    grid = (B, n_l, TOPK)
    out = pl.pallas_call(
        _moe_body,
        grid_spec=pltpu.PrefetchScalarGridSpec(
            num_scalar_prefetch=2,
            grid=grid,
            in_specs=[
                pl.BlockSpec(
                    (1, 1, C, LB),
                    lambda b, l, t, sel_ref, w_ref: (b, sel_ref[b, t], 0, l),
                ),
                pl.BlockSpec(
                    (1, C, C),
                    lambda b, l, t, sel_ref, w_ref: (sel_ref[b, t], 0, 0),
                ),
            ],
            out_specs=pl.BlockSpec(
                (1, C, LB),
                lambda b, l, t, sel_ref, w_ref: (b, 0, l),
            ),
        ),
        out_shape=jax.ShapeDtypeStruct((B, C, L), jnp.float32),
    )(sel, w, xs, expert_weights)
    return out


# speedup vs baseline: 9.4274x; 1.6289x over previous
import jax
import jax.numpy as jnp
from jax.experimental import pallas as pl
from jax.experimental.pallas import tpu as pltpu

B, K, C, L = 16, 8, 128, 4096
TOPK = 4
LB = 4096


def _routing_body(s_ref, sel_ref, w_ref):
    s = s_ref[...]
    iota = jax.lax.broadcasted_iota(jnp.int32, (B, K), 1)
    cur = s
    vals = []
    idxs = []
    for _ in range(TOPK):
        m = jnp.max(cur, axis=1, keepdims=True)
        is_m = cur == m
        idx = jnp.min(jnp.where(is_m, iota, K), axis=1, keepdims=True)
        vals.append(m)
        idxs.append(idx)
        cur = jnp.where(iota == idx, -jnp.inf, cur)
    v = jnp.concatenate(vals, axis=1)
    i = jnp.concatenate(idxs, axis=1)
    w = v / (jnp.sum(v, axis=1, keepdims=True) + 1e-8)
    sel_ref[...] = i
    w_ref[...] = w


def _moe_body(sel_ref, w_ref, *refs):
    x_refs = refs[:TOPK]
    ew_refs = refs[TOPK:2 * TOPK]
    o_ref = refs[2 * TOPK]
    b = pl.program_id(0)
    acc = None
    for t in range(TOPK):
        w = w_ref[b, t]
        ew = (ew_refs[t][0] * w).astype(jnp.bfloat16)
        x = x_refs[t][0, 0].astype(jnp.bfloat16)
        d = jax.lax.dot_general(
            ew, x,
            dimension_numbers=(((0,), (0,)), ((), ())),
            preferred_element_type=jnp.float32,
        )
        acc = d if acc is None else acc + d
    o_ref[0] = acc


@jax.jit
def kernel(xs, scores, expert_weights):
    sel, w = pl.pallas_call(
        _routing_body,
        out_shape=(
            jax.ShapeDtypeStruct((B, TOPK), jnp.int32),
            jax.ShapeDtypeStruct((B, TOPK), jnp.float32),
        ),
    )(scores)

    n_l = L // LB
    grid = (B, n_l)

    def x_map(t):
        return lambda b, l, sel_ref, w_ref: (b, sel_ref[b, t], 0, l)

    def ew_map(t):
        return lambda b, l, sel_ref, w_ref: (sel_ref[b, t], 0, 0)

    in_specs = (
        [pl.BlockSpec((1, 1, C, LB), x_map(t)) for t in range(TOPK)]
        + [pl.BlockSpec((1, C, C), ew_map(t)) for t in range(TOPK)]
    )
    out = pl.pallas_call(
        _moe_body,
        grid_spec=pltpu.PrefetchScalarGridSpec(
            num_scalar_prefetch=2,
            grid=grid,
            in_specs=in_specs,
            out_specs=pl.BlockSpec(
                (1, C, LB),
                lambda b, l, sel_ref, w_ref: (b, 0, l),
            ),
        ),
        out_shape=jax.ShapeDtypeStruct((B, C, L), jnp.float32),
    )(sel, w, *([xs] * TOPK), *([expert_weights] * TOPK))
    return out


# speedup vs baseline: 9.5358x; 1.0115x over previous
import jax
import jax.numpy as jnp
from jax.experimental import pallas as pl
from jax.experimental.pallas import tpu as pltpu

B, K, C, L = 16, 8, 128, 4096
TOPK = 4
LB = 4096


def _routing_body(s_ref, sel_ref, w_ref):
    s = s_ref[...]
    iota = jax.lax.broadcasted_iota(jnp.int32, (B, K), 1)
    cur = s
    vals = []
    idxs = []
    for _ in range(TOPK):
        m = jnp.max(cur, axis=1, keepdims=True)
        is_m = cur == m
        idx = jnp.min(jnp.where(is_m, iota, K), axis=1, keepdims=True)
        vals.append(m)
        idxs.append(idx)
        cur = jnp.where(iota == idx, -jnp.inf, cur)
    v = jnp.concatenate(vals, axis=1)
    i = jnp.concatenate(idxs, axis=1)
    w = v / (jnp.sum(v, axis=1, keepdims=True) + 1e-8)
    sel_ref[...] = i
    w_ref[...] = w


def _moe_body(sel_ref, w_ref, *refs):
    x_refs = refs[:TOPK]
    ew_ref = refs[TOPK]
    o_ref = refs[TOPK + 1]
    b = pl.program_id(0)
    acc = None
    for t in range(TOPK):
        w = w_ref[b, t]
        e = sel_ref[b, t]
        ew = (ew_ref[e] * w).astype(jnp.bfloat16)
        x = x_refs[t][0, 0].astype(jnp.bfloat16)
        d = jax.lax.dot_general(
            ew, x,
            dimension_numbers=(((0,), (0,)), ((), ())),
            preferred_element_type=jnp.float32,
        )
        acc = d if acc is None else acc + d
    o_ref[0] = acc


@jax.jit
def kernel(xs, scores, expert_weights):
    sel, w = pl.pallas_call(
        _routing_body,
        out_shape=(
            jax.ShapeDtypeStruct((B, TOPK), jnp.int32),
            jax.ShapeDtypeStruct((B, TOPK), jnp.float32),
        ),
    )(scores)

    n_l = L // LB
    grid = (B, n_l)

    def x_map(t):
        return lambda b, l, sel_ref, w_ref: (b, sel_ref[b, t], 0, l)

    in_specs = (
        [pl.BlockSpec((1, 1, C, LB), x_map(t)) for t in range(TOPK)]
        + [pl.BlockSpec((K, C, C), lambda b, l, sel_ref, w_ref: (0, 0, 0))]
    )
    out = pl.pallas_call(
        _moe_body,
        grid_spec=pltpu.PrefetchScalarGridSpec(
            num_scalar_prefetch=2,
            grid=grid,
            in_specs=in_specs,
            out_specs=pl.BlockSpec(
                (1, C, LB),
                lambda b, l, sel_ref, w_ref: (b, 0, l),
            ),
        ),
        out_shape=jax.ShapeDtypeStruct((B, C, L), jnp.float32),
    )(sel, w, *([xs] * TOPK), expert_weights)
    return out
